# Initial kernel scaffold; baseline (speedup 1.0000x reference)
import jax
import jax.numpy as jnp
from jax.experimental import pallas as pl


def kernel(x, edge_index, params):
    raise NotImplementedError("write your pallas kernel here")


# speedup vs baseline: 3.8920x; 3.8920x over previous
import functools

import jax
import jax.numpy as jnp
from jax import lax
from jax.experimental import pallas as pl
from jax.experimental.pallas import tpu as pltpu
from jax.experimental.pallas import tpu_sc as plsc

NN = 50000
NE = 800000
D = 64
HALF = 32

NC, NS, L = 2, 16, 16
BLK = 128
NBLK = NE // BLK
BLK_Q, BLK_R = NBLK // NS, NBLK % NS
NNP = 50048
ROWS_T = NNP // NS
ZROWS = 136

_f32 = jnp.float32
_i32 = jnp.int32


def _sc_layer_call(tab, src, dst):
  mesh = plsc.VectorSubcoreMesh(core_axis_name="c", subcore_axis_name="s",
                                num_cores=NC, num_subcores=NS)
  out_type = jax.ShapeDtypeStruct((NC * NNP, HALF), _f32)

  scratch = [
      pltpu.VMEM((BLK,), _i32),
      pltpu.VMEM((BLK,), _i32),
      pltpu.VMEM((BLK,), _i32),
      pltpu.VMEM((BLK,), _i32),
      pltpu.VMEM((BLK, HALF), _f32),
      pltpu.VMEM((BLK, HALF), _f32),
      pltpu.VMEM((ZROWS, HALF), _f32),
      pltpu.VMEM_SHARED((NNP, HALF), _f32),
      pltpu.SemaphoreType.DMA,
      pltpu.SemaphoreType.DMA,
  ]

  def body(tab_h, src_h, dst_h, s_out, dst_v, src_v, aidx_v, bidx_v,
           arows_v, brows_v, zero_v, acc_sh, sem_a, sem_b):
    c = lax.axis_index("c")
    s = lax.axis_index("s")
    row0 = s * ROWS_T

    def fill_zero(r, _):
      zero_v[r, pl.ds(0, L)] = jnp.zeros((L,), _f32)
      zero_v[r, pl.ds(L, L)] = jnp.zeros((L,), _f32)
      return _
    lax.fori_loop(0, ZROWS, fill_zero, None)

    def zcopy(i, _):
      pltpu.sync_copy(zero_v, acc_sh.at[pl.ds(row0 + i * ZROWS, ZROWS)])
      return _
    lax.fori_loop(0, ROWS_T // ZROWS, zcopy, None)

    plsc.subcore_barrier()

    start = BLK_Q * s + jnp.minimum(s, BLK_R)
    cnt = jnp.where(s < BLK_R, BLK_Q + 1, BLK_Q)

    def do_block(j, _):
      base = (start + j) * BLK
      pltpu.sync_copy(dst_h.at[pl.ds(base, BLK)], dst_v)
      pltpu.sync_copy(src_h.at[pl.ds(base, BLK)], src_v)

      def fill_idx(k, _):
        d = dst_v[pl.ds(k * L, L)]
        aidx_v[pl.ds(k * L, L)] = d * 4 + c
        sr = src_v[pl.ds(k * L, L)]
        bidx_v[pl.ds(k * L, L)] = sr * 4 + (c + 2)
        return _
      lax.fori_loop(0, BLK // L, fill_idx, None)

      da = pltpu.async_copy(tab_h.at[aidx_v], arows_v, sem_a)
      db = pltpu.async_copy(tab_h.at[bidx_v], brows_v, sem_b)
      da.wait()
      db.wait()

      def relu_row(r, _):
        v0 = arows_v[r, pl.ds(0, L)] + brows_v[r, pl.ds(0, L)]
        arows_v[r, pl.ds(0, L)] = jnp.maximum(v0, 0.0)
        v1 = arows_v[r, pl.ds(L, L)] + brows_v[r, pl.ds(L, L)]
        arows_v[r, pl.ds(L, L)] = jnp.maximum(v1, 0.0)
        return _
      lax.fori_loop(0, BLK, relu_row, None)

      pltpu.sync_copy(arows_v, acc_sh.at[dst_v], add=True)
      return _

    lax.fori_loop(0, cnt, do_block, None)

    plsc.subcore_barrier()

    def outcopy(i, _):
      r = row0 + i * ZROWS
      pltpu.sync_copy(acc_sh.at[pl.ds(r, ZROWS)], zero_v)
      pltpu.sync_copy(zero_v, s_out.at[pl.ds(c * NNP + r, ZROWS)])
      return _
    lax.fori_loop(0, ROWS_T // ZROWS, outcopy, None)

  fn = pl.kernel(body, out_type=out_type, mesh=mesh,
                 scratch_types=scratch,
                 compiler_params=pltpu.CompilerParams(use_tc_tiling_on_sc=False))
  return fn(tab, src, dst)


DEGW = 16
NBLK_H = NBLK // 2
DQ, DR = NBLK_H // NS, NBLK_H % NS


def _sc_deg_call(dst):
  mesh = plsc.VectorSubcoreMesh(core_axis_name="c", subcore_axis_name="s",
                                num_cores=NC, num_subcores=NS)
  out_type = (jax.ShapeDtypeStruct((NNP, DEGW), _f32),
              jax.ShapeDtypeStruct((NNP, DEGW), _f32))
  scratch = [
      pltpu.VMEM((BLK,), _i32),
      pltpu.VMEM((BLK, DEGW), _f32),
      pltpu.VMEM((ZROWS, DEGW), _f32),
      pltpu.VMEM_SHARED((NNP, DEGW), _f32),
  ]

  def body(dst_h, out0, out1, dst_v, ones_v, degb_v, deg_sh):
    c = lax.axis_index("c")
    s = lax.axis_index("s")
    row0 = s * ROWS_T

    def fill_ones(r, _):
      ones_v[r, pl.ds(0, L)] = jnp.full((L,), 1.0, _f32)
      return _
    lax.fori_loop(0, BLK, fill_ones, None)

    def fill_zero(r, _):
      degb_v[r, pl.ds(0, L)] = jnp.zeros((L,), _f32)
      return _
    lax.fori_loop(0, ZROWS, fill_zero, None)

    def zcopy(i, _):
      pltpu.sync_copy(degb_v, deg_sh.at[pl.ds(row0 + i * ZROWS, ZROWS)])
      return _
    lax.fori_loop(0, ROWS_T // ZROWS, zcopy, None)

    plsc.subcore_barrier()

    start = c * NBLK_H + DQ * s + jnp.minimum(s, DR)
    cnt = jnp.where(s < DR, DQ + 1, DQ)

    def do_block(j, _):
      base = (start + j) * BLK
      pltpu.sync_copy(dst_h.at[pl.ds(base, BLK)], dst_v)
      pltpu.sync_copy(ones_v, deg_sh.at[dst_v], add=True)
      return _
    lax.fori_loop(0, cnt, do_block, None)

    plsc.subcore_barrier()

    def outcopy(i, _):
      r = row0 + i * ZROWS
      pltpu.sync_copy(deg_sh.at[pl.ds(r, ZROWS)], degb_v)

      @pl.when(c == 0)
      def _():
        pltpu.sync_copy(degb_v, out0.at[pl.ds(r, ZROWS)])

      @pl.when(c == 1)
      def _():
        pltpu.sync_copy(degb_v, out1.at[pl.ds(r, ZROWS)])
      return _
    lax.fori_loop(0, ROWS_T // ZROWS, outcopy, None)

  fn = pl.kernel(body, out_type=out_type, mesh=mesh, scratch_types=scratch,
                 compiler_params=pltpu.CompilerParams(use_tc_tiling_on_sc=False))
  return fn(dst)


R_BLK = 5000


def _dot(a, b):
  return jnp.dot(a, b, preferred_element_type=_f32,
                 precision=lax.Precision.HIGHEST)


def _enc_body(x_ref, encW_ref, encb_ref, w1_ref, b1_ref, h_ref, ab_ref):
  h = _dot(x_ref[...], encW_ref[...]) + encb_ref[...]
  h_ref[...] = h
  ab_ref[...] = _dot(h, w1_ref[...]) + b1_ref[...]


def _enc_call(x8, encW8, encb, w1p, b1p):
  grid = (NN // R_BLK,)
  return pl.pallas_call(
      _enc_body,
      grid=grid,
      in_specs=[
          pl.BlockSpec((R_BLK, 8), lambda i: (i, 0)),
          pl.BlockSpec((8, D), lambda i: (0, 0)),
          pl.BlockSpec((1, D), lambda i: (0, 0)),
          pl.BlockSpec((D, 2 * D), lambda i: (0, 0)),
          pl.BlockSpec((1, 2 * D), lambda i: (0, 0)),
      ],
      out_specs=[
          pl.BlockSpec((R_BLK, D), lambda i: (i, 0)),
          pl.BlockSpec((R_BLK, 2 * D), lambda i: (i, 0)),
      ],
      out_shape=[
          jax.ShapeDtypeStruct((NN, D), _f32),
          jax.ShapeDtypeStruct((NN, 2 * D), _f32),
      ],
  )(x8, encW8, encb, w1p, b1p)


def _post_body(final, h_ref, s_ref, dg0_ref, dg1_ref, eW2_ref, eb2_ref,
               nW1_ref, nb1_ref, nW2_ref, nb2_ref, nxW_ref, nxb_ref, *outs):
  h = h_ref[...]
  deg_col = dg0_ref[:, 0:1] + dg1_ref[:, 0:1]
  agg = (_dot(s_ref[0], eW2_ref[0:HALF, :])
         + _dot(s_ref[1], eW2_ref[HALF:D, :])
         + deg_col * eb2_ref[...])
  t = jnp.maximum(_dot(h, nW1_ref[0:D, :]) + _dot(agg, nW1_ref[D:2 * D, :])
                  + nb1_ref[...], 0.0)
  hn = h + _dot(t, nW2_ref[...]) + nb2_ref[...]
  if final:
    outs[0][...] = _dot(hn, nxW_ref[...]) + nxb_ref[...]
  else:
    outs[0][...] = hn
    outs[1][...] = _dot(hn, nxW_ref[...]) + nxb_ref[...]


def _post_call(h, s2, dg0, dg1, eW2, eb2, nW1, nb1, nW2, nb2, nxW, nxb, final):
  grid = (NN // R_BLK,)
  nxd = nxW.shape[1]
  in_specs = [
      pl.BlockSpec((R_BLK, D), lambda i: (i, 0)),
      pl.BlockSpec((2, R_BLK, HALF), lambda i: (0, i, 0)),
      pl.BlockSpec((R_BLK, DEGW), lambda i: (i, 0)),
      pl.BlockSpec((R_BLK, DEGW), lambda i: (i, 0)),
      pl.BlockSpec((D, D), lambda i: (0, 0)),
      pl.BlockSpec((1, D), lambda i: (0, 0)),
      pl.BlockSpec((2 * D, D), lambda i: (0, 0)),
      pl.BlockSpec((1, D), lambda i: (0, 0)),
      pl.BlockSpec((D, D), lambda i: (0, 0)),
      pl.BlockSpec((1, D), lambda i: (0, 0)),
      pl.BlockSpec((D, nxd), lambda i: (0, 0)),
      pl.BlockSpec((1, nxd), lambda i: (0, 0)),
  ]
  if final:
    out_specs = [pl.BlockSpec((R_BLK, nxd), lambda i: (i, 0))]
    out_shape = [jax.ShapeDtypeStruct((NN, nxd), _f32)]
  else:
    out_specs = [
        pl.BlockSpec((R_BLK, D), lambda i: (i, 0)),
        pl.BlockSpec((R_BLK, nxd), lambda i: (i, 0)),
    ]
    out_shape = [
        jax.ShapeDtypeStruct((NN, D), _f32),
        jax.ShapeDtypeStruct((NN, nxd), _f32),
    ]
  return pl.pallas_call(
      functools.partial(_post_body, final),
      grid=grid,
      in_specs=in_specs,
      out_specs=out_specs,
      out_shape=out_shape,
  )(h, s2, dg0, dg1, eW2, eb2, nW1, nb1, nW2, nb2, nxW, nxb)


def kernel(x, edge_index, params):
  p = params
  src = edge_index[0]
  dst = edge_index[1]

  x8 = jnp.pad(x, ((0, 0), (0, 1)))
  encW8 = jnp.pad(p['enc_W'], ((0, 1), (0, 0)))
  encb = p['enc_b'][None, :]
  w1p = []
  b1p = []
  for l in range(3):
    eW1 = p[f'l{l}_eW1']
    w1p.append(jnp.concatenate([eW1[:D], eW1[D:]], axis=1))
    b1p.append(jnp.concatenate([p[f'l{l}_eb1'], jnp.zeros((D,), _f32)])[None, :])
  decW8 = jnp.pad(p['dec_W'], ((0, 0), (0, 2)))
  decb8 = jnp.pad(p['dec_b'], (0, 2))[None, :]

  h, ab = _enc_call(x8, encW8, encb, w1p[0], b1p[0])
  dg0, dg1 = _sc_deg_call(dst)

  for l in range(3):
    tab = ab.reshape(4 * NN, HALF)
    s_flat = _sc_layer_call(tab, src, dst)
    s2 = s_flat.reshape(2, NNP, HALF)
    final = (l == 2)
    if final:
      out, = _post_call(h, s2, dg0, dg1, p[f'l{l}_eW2'], p[f'l{l}_eb2'][None, :],
                        p[f'l{l}_nW1'], p[f'l{l}_nb1'][None, :],
                        p[f'l{l}_nW2'], p[f'l{l}_nb2'][None, :],
                        decW8, decb8, True)
    else:
      h, ab = _post_call(h, s2, dg0, dg1, p[f'l{l}_eW2'], p[f'l{l}_eb2'][None, :],
                         p[f'l{l}_nW1'], p[f'l{l}_nb1'][None, :],
                         p[f'l{l}_nW2'], p[f'l{l}_nb2'][None, :],
                         w1p[l + 1], b1p[l + 1], False)
  return out[:, :6]


# speedup vs baseline: 5.8380x; 1.5000x over previous
import functools

import jax
import jax.numpy as jnp
from jax import lax
from jax.experimental import pallas as pl
from jax.experimental.pallas import tpu as pltpu
from jax.experimental.pallas import tpu_sc as plsc

NN = 50000
NE = 800000
D = 64
HALF = 32

NC, NS, L = 2, 16, 16
BLK = 128
NBLK = NE // BLK
NNP = 50048
ROWS_T = NNP // NS
ZROWS = 136

_f32 = jnp.float32
_i32 = jnp.int32


def _sc_layer_call(tab, src, dst):
  mesh = plsc.VectorSubcoreMesh(core_axis_name="c", subcore_axis_name="s",
                                num_cores=NC, num_subcores=NS)
  out_type = jax.ShapeDtypeStruct((NC * NNP, HALF), _f32)

  scratch = [
      pltpu.VMEM((2 * BLK,), _i32),
      pltpu.VMEM((2 * BLK,), _i32),
      pltpu.VMEM((BLK,), _i32),
      pltpu.VMEM((BLK,), _i32),
      pltpu.VMEM((BLK,), _i32),
      pltpu.VMEM((BLK,), _i32),
      pltpu.VMEM((BLK,), _i32),
      pltpu.VMEM((BLK,), _i32),
      pltpu.VMEM((BLK,), _i32),
      pltpu.VMEM((BLK,), _i32),
      pltpu.VMEM((BLK, HALF), _f32),
      pltpu.VMEM((BLK, HALF), _f32),
      pltpu.VMEM((BLK, HALF), _f32),
      pltpu.VMEM((BLK, HALF), _f32),
      pltpu.VMEM((BLK, HALF), _f32),
      pltpu.VMEM((BLK, HALF), _f32),
      pltpu.VMEM((ZROWS, HALF), _f32),
      pltpu.VMEM_SHARED((NNP, HALF), _f32),
      pltpu.SemaphoreType.DMA,
      pltpu.SemaphoreType.DMA,
      pltpu.SemaphoreType.DMA,
      pltpu.SemaphoreType.DMA,
      pltpu.SemaphoreType.DMA,
      pltpu.SemaphoreType.DMA,
  ]

  def body(tab_h, src_h, dst_h, s_out, dst_c, src_c, aidx0, aidx1,
           bidx0, bidx1, dstx0, dstx1, sdst0, sdst1, arows0, arows1,
           brows0, brows1, srows0, srows1, zero_v, acc_sh,
           sga0, sga1, sgb0, sgb1, ssc0, ssc1):
    aidx = (aidx0, aidx1)
    bidx = (bidx0, bidx1)
    dstx = (dstx0, dstx1)
    sdst = (sdst0, sdst1)
    arows = (arows0, arows1)
    brows = (brows0, brows1)
    srows = (srows0, srows1)
    sga = (sga0, sga1)
    sgb = (sgb0, sgb1)
    ssc = (ssc0, ssc1)

    c = lax.axis_index("c")
    s = lax.axis_index("s")
    row0 = s * ROWS_T
    aoff = c * NN
    boff = (2 + c) * NN

    @plsc.parallel_loop(0, ZROWS, unroll=4)
    def _(r):
      zero_v[r, pl.ds(0, L)] = jnp.zeros((L,), _f32)
      zero_v[r, pl.ds(L, L)] = jnp.zeros((L,), _f32)

    def zcopy(i, _):
      pltpu.sync_copy(zero_v, acc_sh.at[pl.ds(row0 + i * ZROWS, ZROWS)])
      return _
    lax.fori_loop(0, ROWS_T // ZROWS, zcopy, None)

    plsc.subcore_barrier()

    start = 390 * s + 2 * jnp.minimum(s, 5)
    cnt = jnp.where(s < 5, 392, 390)

    def load_chunk(blk):
      base = (start + blk) * BLK
      pltpu.sync_copy(dst_h.at[pl.ds(base, 2 * BLK)], dst_c)
      pltpu.sync_copy(src_h.at[pl.ds(base, 2 * BLK)], src_c)

    def fill_block(b, coff):
      @plsc.parallel_loop(0, BLK // L, unroll=2)
      def _(k):
        d = dst_c[pl.ds(coff + k * L, L)]
        dstx[b][pl.ds(k * L, L)] = d
        aidx[b][pl.ds(k * L, L)] = d + aoff
        sr = src_c[pl.ds(coff + k * L, L)]
        bidx[b][pl.ds(k * L, L)] = sr + boff

    def start_gathers(b):
      pltpu.async_copy(tab_h.at[aidx[b]], arows[b], sga[b])
      pltpu.async_copy(tab_h.at[bidx[b]], brows[b], sgb[b])

    load_chunk(0)
    fill_block(0, 0)
    fill_block(1, BLK)
    start_gathers(0)
    start_gathers(1)

    def pair(t, _):
      for b in (0, 1):
        j = 2 * t + b
        pltpu.make_async_copy(tab_h.at[aidx[b]], arows[b], sga[b]).wait()
        pltpu.make_async_copy(tab_h.at[bidx[b]], brows[b], sgb[b]).wait()

        @pl.when(j >= 2)
        def _():
          pltpu.make_async_copy(srows[b], acc_sh.at[sdst[b]], ssc[b]).wait()

        @plsc.parallel_loop(0, BLK, unroll=4)
        def _(r):
          v0 = arows[b][r, pl.ds(0, L)] + brows[b][r, pl.ds(0, L)]
          srows[b][r, pl.ds(0, L)] = jnp.maximum(v0, 0.0)
          v1 = arows[b][r, pl.ds(L, L)] + brows[b][r, pl.ds(L, L)]
          srows[b][r, pl.ds(L, L)] = jnp.maximum(v1, 0.0)

        @plsc.parallel_loop(0, BLK // L, unroll=2)
        def _(k):
          sdst[b][pl.ds(k * L, L)] = dstx[b][pl.ds(k * L, L)]

        pltpu.async_copy(srows[b], acc_sh.at[sdst[b]], ssc[b], add=True)

        jn = j + 2

        @pl.when(jn < cnt)
        def _():
          if b == 0:
            load_chunk(jn)
          fill_block(b, b * BLK)
          start_gathers(b)
      return _

    lax.fori_loop(0, cnt // 2, pair, None)

    for b in (0, 1):
      pltpu.make_async_copy(srows[b], acc_sh.at[sdst[b]], ssc[b]).wait()

    plsc.subcore_barrier()

    def outcopy(i, _):
      r = row0 + i * ZROWS
      pltpu.sync_copy(acc_sh.at[pl.ds(r, ZROWS)], zero_v)
      pltpu.sync_copy(zero_v, s_out.at[pl.ds(c * NNP + r, ZROWS)])
      return _
    lax.fori_loop(0, ROWS_T // ZROWS, outcopy, None)

  fn = pl.kernel(body, out_type=out_type, mesh=mesh,
                 scratch_types=scratch,
                 compiler_params=pltpu.CompilerParams(use_tc_tiling_on_sc=False))
  return fn(tab, src, dst)


DEGW = 16
NBLK_H = NBLK // 2
DQ, DR = NBLK_H // NS, NBLK_H % NS


def _sc_deg_call(dst):
  mesh = plsc.VectorSubcoreMesh(core_axis_name="c", subcore_axis_name="s",
                                num_cores=NC, num_subcores=NS)
  out_type = (jax.ShapeDtypeStruct((NNP, DEGW), _f32),
              jax.ShapeDtypeStruct((NNP, DEGW), _f32))
  scratch = [
      pltpu.VMEM((BLK,), _i32),
      pltpu.VMEM((BLK, DEGW), _f32),
      pltpu.VMEM((ZROWS, DEGW), _f32),
      pltpu.VMEM_SHARED((NNP, DEGW), _f32),
  ]

  def body(dst_h, out0, out1, dst_v, ones_v, degb_v, deg_sh):
    c = lax.axis_index("c")
    s = lax.axis_index("s")
    row0 = s * ROWS_T

    @plsc.parallel_loop(0, BLK, unroll=4)
    def _(r):
      ones_v[r, pl.ds(0, L)] = jnp.full((L,), 1.0, _f32)

    @plsc.parallel_loop(0, ZROWS, unroll=4)
    def _(r):
      degb_v[r, pl.ds(0, L)] = jnp.zeros((L,), _f32)

    def zcopy(i, _):
      pltpu.sync_copy(degb_v, deg_sh.at[pl.ds(row0 + i * ZROWS, ZROWS)])
      return _
    lax.fori_loop(0, ROWS_T // ZROWS, zcopy, None)

    plsc.subcore_barrier()

    start = c * NBLK_H + DQ * s + jnp.minimum(s, DR)
    cnt = jnp.where(s < DR, DQ + 1, DQ)

    def do_block(j, _):
      base = (start + j) * BLK
      pltpu.sync_copy(dst_h.at[pl.ds(base, BLK)], dst_v)
      pltpu.sync_copy(ones_v, deg_sh.at[dst_v], add=True)
      return _
    lax.fori_loop(0, cnt, do_block, None)

    plsc.subcore_barrier()

    def outcopy(i, _):
      r = row0 + i * ZROWS
      pltpu.sync_copy(deg_sh.at[pl.ds(r, ZROWS)], degb_v)

      @pl.when(c == 0)
      def _():
        pltpu.sync_copy(degb_v, out0.at[pl.ds(r, ZROWS)])

      @pl.when(c == 1)
      def _():
        pltpu.sync_copy(degb_v, out1.at[pl.ds(r, ZROWS)])
      return _
    lax.fori_loop(0, ROWS_T // ZROWS, outcopy, None)

  fn = pl.kernel(body, out_type=out_type, mesh=mesh, scratch_types=scratch,
                 compiler_params=pltpu.CompilerParams(use_tc_tiling_on_sc=False))
  return fn(dst)


R_BLK = 5000


def _dot(a, b):
  return jnp.dot(a, b, preferred_element_type=_f32,
                 precision=lax.Precision.HIGHEST)


def _store_tab(ab4_ref, ab):
  ab4_ref[0] = ab[:, 0:HALF]
  ab4_ref[1] = ab[:, HALF:D]
  ab4_ref[2] = ab[:, D:D + HALF]
  ab4_ref[3] = ab[:, D + HALF:2 * D]


def _enc_body(x_ref, encW_ref, encb_ref, w1_ref, b1_ref, h_ref, ab4_ref):
  h = _dot(x_ref[...], encW_ref[...]) + encb_ref[...]
  h_ref[...] = h
  _store_tab(ab4_ref, _dot(h, w1_ref[...]) + b1_ref[...])


def _enc_call(x8, encW8, encb, w1p, b1p):
  grid = (NN // R_BLK,)
  return pl.pallas_call(
      _enc_body,
      grid=grid,
      in_specs=[
          pl.BlockSpec((R_BLK, 8), lambda i: (i, 0)),
          pl.BlockSpec((8, D), lambda i: (0, 0)),
          pl.BlockSpec((1, D), lambda i: (0, 0)),
          pl.BlockSpec((D, 2 * D), lambda i: (0, 0)),
          pl.BlockSpec((1, 2 * D), lambda i: (0, 0)),
      ],
      out_specs=[
          pl.BlockSpec((R_BLK, D), lambda i: (i, 0)),
          pl.BlockSpec((4, R_BLK, HALF), lambda i: (0, i, 0)),
      ],
      out_shape=[
          jax.ShapeDtypeStruct((NN, D), _f32),
          jax.ShapeDtypeStruct((4, NN, HALF), _f32),
      ],
  )(x8, encW8, encb, w1p, b1p)


def _post_body(final, h_ref, s_ref, dg0_ref, dg1_ref, eW2_ref, eb2_ref,
               nW1_ref, nb1_ref, nW2_ref, nb2_ref, nxW_ref, nxb_ref, *outs):
  h = h_ref[...]
  deg_col = dg0_ref[:, 0:1] + dg1_ref[:, 0:1]
  agg = (_dot(s_ref[0], eW2_ref[0:HALF, :])
         + _dot(s_ref[1], eW2_ref[HALF:D, :])
         + deg_col * eb2_ref[...])
  t = jnp.maximum(_dot(h, nW1_ref[0:D, :]) + _dot(agg, nW1_ref[D:2 * D, :])
                  + nb1_ref[...], 0.0)
  hn = h + _dot(t, nW2_ref[...]) + nb2_ref[...]
  if final:
    outs[0][...] = _dot(hn, nxW_ref[...]) + nxb_ref[...]
  else:
    outs[0][...] = hn
    _store_tab(outs[1], _dot(hn, nxW_ref[...]) + nxb_ref[...])


def _post_call(h, s2, dg0, dg1, eW2, eb2, nW1, nb1, nW2, nb2, nxW, nxb, final):
  grid = (NN // R_BLK,)
  nxd = nxW.shape[1]
  in_specs = [
      pl.BlockSpec((R_BLK, D), lambda i: (i, 0)),
      pl.BlockSpec((2, R_BLK, HALF), lambda i: (0, i, 0)),
      pl.BlockSpec((R_BLK, DEGW), lambda i: (i, 0)),
      pl.BlockSpec((R_BLK, DEGW), lambda i: (i, 0)),
      pl.BlockSpec((D, D), lambda i: (0, 0)),
      pl.BlockSpec((1, D), lambda i: (0, 0)),
      pl.BlockSpec((2 * D, D), lambda i: (0, 0)),
      pl.BlockSpec((1, D), lambda i: (0, 0)),
      pl.BlockSpec((D, D), lambda i: (0, 0)),
      pl.BlockSpec((1, D), lambda i: (0, 0)),
      pl.BlockSpec((D, nxd), lambda i: (0, 0)),
      pl.BlockSpec((1, nxd), lambda i: (0, 0)),
  ]
  if final:
    out_specs = [pl.BlockSpec((R_BLK, nxd), lambda i: (i, 0))]
    out_shape = [jax.ShapeDtypeStruct((NN, nxd), _f32)]
  else:
    out_specs = [
        pl.BlockSpec((R_BLK, D), lambda i: (i, 0)),
        pl.BlockSpec((4, R_BLK, HALF), lambda i: (0, i, 0)),
    ]
    out_shape = [
        jax.ShapeDtypeStruct((NN, D), _f32),
        jax.ShapeDtypeStruct((4, NN, HALF), _f32),
    ]
  return pl.pallas_call(
      functools.partial(_post_body, final),
      grid=grid,
      in_specs=in_specs,
      out_specs=out_specs,
      out_shape=out_shape,
  )(h, s2, dg0, dg1, eW2, eb2, nW1, nb1, nW2, nb2, nxW, nxb)


def kernel(x, edge_index, params):
  p = params
  src = edge_index[0]
  dst = edge_index[1]

  x8 = jnp.pad(x, ((0, 0), (0, 1)))
  encW8 = jnp.pad(p['enc_W'], ((0, 1), (0, 0)))
  encb = p['enc_b'][None, :]
  w1p = []
  b1p = []
  for l in range(3):
    eW1 = p[f'l{l}_eW1']
    w1p.append(jnp.concatenate([eW1[:D], eW1[D:]], axis=1))
    b1p.append(jnp.concatenate([p[f'l{l}_eb1'], jnp.zeros((D,), _f32)])[None, :])
  decW8 = jnp.pad(p['dec_W'], ((0, 0), (0, 2)))
  decb8 = jnp.pad(p['dec_b'], (0, 2))[None, :]

  h, ab4 = _enc_call(x8, encW8, encb, w1p[0], b1p[0])
  dg0, dg1 = _sc_deg_call(dst)

  for l in range(3):
    tab = ab4.reshape(4 * NN, HALF)
    s_flat = _sc_layer_call(tab, src, dst)
    s2 = s_flat.reshape(2, NNP, HALF)
    final = (l == 2)
    if final:
      out, = _post_call(h, s2, dg0, dg1, p[f'l{l}_eW2'], p[f'l{l}_eb2'][None, :],
                        p[f'l{l}_nW1'], p[f'l{l}_nb1'][None, :],
                        p[f'l{l}_nW2'], p[f'l{l}_nb2'][None, :],
                        decW8, decb8, True)
    else:
      h, ab4 = _post_call(h, s2, dg0, dg1, p[f'l{l}_eW2'], p[f'l{l}_eb2'][None, :],
                          p[f'l{l}_nW1'], p[f'l{l}_nb1'][None, :],
                          p[f'l{l}_nW2'], p[f'l{l}_nb2'][None, :],
                          w1p[l + 1], b1p[l + 1], False)
  return out[:, :6]
